# baseline (speedup 1.0000x reference)
import functools

import jax
import jax.numpy as jnp
from jax.experimental import pallas as pl
from jax.experimental.pallas import tpu as pltpu

BM = 1024
PFT = 512


def _ffn_kernel(x_ref, w1_ref, b1_ref, w2_ref, b2_ref, out_ref, hb):
    xb = x_ref[...].astype(jnp.bfloat16)
    pf = w1_ref.shape[1]
    for j in range(pf // PFT):
        sl = pl.ds(j * PFT, PFT)
        h = jnp.dot(xb, w1_ref[:, sl], preferred_element_type=jnp.float32)
        h = jnp.maximum(h + b1_ref[:, sl], 0.0)
        hb[:, sl] = h.astype(jnp.bfloat16)
    out = jnp.dot(hb[...], w2_ref[...], preferred_element_type=jnp.float32)
    out_ref[...] = out + b2_ref[...]


@functools.partial(jax.jit, static_argnames=())
def kernel(x, W1, b1, W2, b2):
    B, S, H = x.shape
    PF = W1.shape[1]
    M = B * S
    x2 = x.reshape(M, H)
    w1b = W1.astype(jnp.bfloat16)
    w2b = W2.astype(jnp.bfloat16)
    b1r = b1.reshape(1, PF)
    b2r = b2.reshape(1, H)

    out = pl.pallas_call(
        _ffn_kernel,
        grid=(M // BM,),
        in_specs=[
            pl.BlockSpec((BM, H), lambda i: (i, 0)),
            pl.BlockSpec((H, PF), lambda i: (0, 0)),
            pl.BlockSpec((1, PF), lambda i: (0, 0)),
            pl.BlockSpec((PF, H), lambda i: (0, 0)),
            pl.BlockSpec((1, H), lambda i: (0, 0)),
        ],
        out_specs=pl.BlockSpec((BM, H), lambda i: (i, 0)),
        out_shape=jax.ShapeDtypeStruct((M, H), jnp.float32),
        scratch_shapes=[
            pltpu.VMEM((BM, PF), jnp.bfloat16),
        ],
        compiler_params=pltpu.CompilerParams(
            dimension_semantics=("arbitrary",),
            vmem_limit_bytes=63 * 1024 * 1024,
        ),
    )(x2, w1b, b1r, w2b, b2r)
    return out.reshape(B, S, H)


# speedup vs baseline: 1.1098x; 1.1098x over previous
import functools

import jax
import jax.numpy as jnp
from jax.experimental import pallas as pl
from jax.experimental.pallas import tpu as pltpu

BM = 1024
PFT = 512
LOOK = 3


def _ffn_kernel(x_ref, w1_hbm, b1_ref, w2_hbm, b2_ref, out_ref,
                w1t, w2bf, hb, land1, land2, sem1, sem2):
    i = pl.program_id(0)
    nsteps = pl.num_programs(0)
    n_tiles = w1_hbm.shape[1] // PFT

    def _copies(c):
        p = c % LOOK
        return (
            pltpu.make_async_copy(
                w1_hbm.at[:, pl.ds(c * PFT, PFT)], land1.at[p], sem1.at[p]),
            pltpu.make_async_copy(
                w2_hbm.at[pl.ds(c * PFT, PFT), :], land2.at[p], sem2.at[p]),
        )

    def _start(c):
        for cp in _copies(c):
            cp.start()

    def _wait(c):
        for cp in _copies(c):
            cp.wait()

    @pl.when(i == 0)
    def _():
        for c in range(LOOK):
            _start(c)

    xb = x_ref[...].astype(jnp.bfloat16)
    for j in range(n_tiles):
        _wait(j)
        s2 = j % 2
        sl = pl.ds(j * PFT, PFT)
        w1t[s2] = land1[j % LOOK].astype(jnp.bfloat16)
        w2bf[sl, :] = land2[j % LOOK].astype(jnp.bfloat16)
        nxt = j + LOOK
        if nxt < n_tiles:
            _start(nxt)
        else:
            @pl.when(i < nsteps - 1)
            def _(nxt=nxt):
                _start(nxt - n_tiles)
        h = jnp.dot(xb, w1t[s2], preferred_element_type=jnp.float32)
        h = jnp.maximum(h + b1_ref[:, sl], 0.0)
        hb[:, sl] = h.astype(jnp.bfloat16)
    out = jnp.dot(hb[...], w2bf[...], preferred_element_type=jnp.float32)
    out_ref[...] = out + b2_ref[...]


@functools.partial(jax.jit, static_argnames=())
def kernel(x, W1, b1, W2, b2):
    B, S, H = x.shape
    PF = W1.shape[1]
    M = B * S
    x2 = x.reshape(M, H)
    b1r = b1.reshape(1, PF)
    b2r = b2.reshape(1, H)

    out = pl.pallas_call(
        _ffn_kernel,
        grid=(M // BM,),
        in_specs=[
            pl.BlockSpec((BM, H), lambda i: (i, 0)),
            pl.BlockSpec(memory_space=pl.ANY),
            pl.BlockSpec((1, PF), lambda i: (0, 0)),
            pl.BlockSpec(memory_space=pl.ANY),
            pl.BlockSpec((1, H), lambda i: (0, 0)),
        ],
        out_specs=pl.BlockSpec((BM, H), lambda i: (i, 0)),
        out_shape=jax.ShapeDtypeStruct((M, H), jnp.float32),
        scratch_shapes=[
            pltpu.VMEM((2, H, PFT), jnp.bfloat16),
            pltpu.VMEM((PF, H), jnp.bfloat16),
            pltpu.VMEM((BM, PF), jnp.bfloat16),
            pltpu.VMEM((LOOK, H, PFT), jnp.float32),
            pltpu.VMEM((LOOK, PFT, H), jnp.float32),
            pltpu.SemaphoreType.DMA((LOOK,)),
            pltpu.SemaphoreType.DMA((LOOK,)),
        ],
        compiler_params=pltpu.CompilerParams(
            dimension_semantics=("arbitrary",),
            vmem_limit_bytes=63 * 1024 * 1024,
        ),
    )(x2, W1, b1r, W2, b2r)
    return out.reshape(B, S, H)


# speedup vs baseline: 1.1202x; 1.0094x over previous
import functools

import jax
import jax.numpy as jnp
from jax.experimental import pallas as pl
from jax.experimental.pallas import tpu as pltpu

BM = 1024
PFT = 512
LOOK = 4


def _ffn_kernel(x_ref, w1_hbm, b1_ref, w2_hbm, b2_ref, out_ref,
                w1t, w2bf, hb, land1, land2, sem1, sem2):
    i = pl.program_id(0)
    nsteps = pl.num_programs(0)
    n_tiles = w1_hbm.shape[1] // PFT

    def _copies(c):
        p = c % LOOK
        return (
            pltpu.make_async_copy(
                w1_hbm.at[:, pl.ds(c * PFT, PFT)], land1.at[p], sem1.at[p]),
            pltpu.make_async_copy(
                w2_hbm.at[pl.ds(c * PFT, PFT), :], land2.at[p], sem2.at[p]),
        )

    def _start(c):
        for cp in _copies(c):
            cp.start()

    def _wait(c):
        for cp in _copies(c):
            cp.wait()

    @pl.when(i == 0)
    def _():
        for c in range(LOOK):
            _start(c)

    xb = x_ref[...].astype(jnp.bfloat16)
    for j in range(n_tiles):
        _wait(j)
        s2 = j % 2
        sl = pl.ds(j * PFT, PFT)
        w1t[s2] = land1[j % LOOK].astype(jnp.bfloat16)
        w2bf[sl, :] = land2[j % LOOK].astype(jnp.bfloat16)
        nxt = j + LOOK
        if nxt < n_tiles:
            _start(nxt)
        else:
            @pl.when(i < nsteps - 1)
            def _(nxt=nxt):
                _start(nxt - n_tiles)
        h = jnp.dot(xb, w1t[s2], preferred_element_type=jnp.float32)
        h = jnp.maximum(h + b1_ref[:, sl], 0.0)
        hb[:, sl] = h.astype(jnp.bfloat16)
    out = jnp.dot(hb[...], w2bf[...], preferred_element_type=jnp.float32)
    out_ref[...] = out + b2_ref[...]


@functools.partial(jax.jit, static_argnames=())
def kernel(x, W1, b1, W2, b2):
    B, S, H = x.shape
    PF = W1.shape[1]
    M = B * S
    x2 = x.reshape(M, H)
    b1r = b1.reshape(1, PF)
    b2r = b2.reshape(1, H)

    out = pl.pallas_call(
        _ffn_kernel,
        grid=(M // BM,),
        in_specs=[
            pl.BlockSpec((BM, H), lambda i: (i, 0)),
            pl.BlockSpec(memory_space=pl.ANY),
            pl.BlockSpec((1, PF), lambda i: (0, 0)),
            pl.BlockSpec(memory_space=pl.ANY),
            pl.BlockSpec((1, H), lambda i: (0, 0)),
        ],
        out_specs=pl.BlockSpec((BM, H), lambda i: (i, 0)),
        out_shape=jax.ShapeDtypeStruct((M, H), jnp.float32),
        scratch_shapes=[
            pltpu.VMEM((2, H, PFT), jnp.bfloat16),
            pltpu.VMEM((PF, H), jnp.bfloat16),
            pltpu.VMEM((BM, PF), jnp.bfloat16),
            pltpu.VMEM((LOOK, H, PFT), jnp.float32),
            pltpu.VMEM((LOOK, PFT, H), jnp.float32),
            pltpu.SemaphoreType.DMA((LOOK,)),
            pltpu.SemaphoreType.DMA((LOOK,)),
        ],
        compiler_params=pltpu.CompilerParams(
            dimension_semantics=("arbitrary",),
            vmem_limit_bytes=63 * 1024 * 1024,
        ),
    )(x2, W1, b1r, W2, b2r)
    return out.reshape(B, S, H)


# speedup vs baseline: 1.1259x; 1.0051x over previous
import functools

import jax
import jax.numpy as jnp
from jax.experimental import pallas as pl
from jax.experimental.pallas import tpu as pltpu

BM = 1024
PFT = 512


def _ffn_kernel(x_ref, w1_hbm, b1_ref, w2_hbm, b2_ref, out_ref,
                w1bf, w2bf, hb, land1, land2, sem1, sem2):
    i = pl.program_id(0)
    n_tiles = w1_hbm.shape[1] // PFT

    def _start(j):
        p = j % 2
        pltpu.make_async_copy(
            w1_hbm.at[:, pl.ds(j * PFT, PFT)], land1.at[p], sem1.at[p]
        ).start()
        pltpu.make_async_copy(
            w2_hbm.at[pl.ds(j * PFT, PFT), :], land2.at[p], sem2.at[p]
        ).start()

    def _wait_and_cast(j):
        p = j % 2
        pltpu.make_async_copy(
            w1_hbm.at[:, pl.ds(j * PFT, PFT)], land1.at[p], sem1.at[p]
        ).wait()
        pltpu.make_async_copy(
            w2_hbm.at[pl.ds(j * PFT, PFT), :], land2.at[p], sem2.at[p]
        ).wait()
        w1bf[:, pl.ds(j * PFT, PFT)] = land1[p].astype(jnp.bfloat16)
        w2bf[pl.ds(j * PFT, PFT), :] = land2[p].astype(jnp.bfloat16)

    @pl.when(i == 0)
    def _():
        _start(0)

    xb = x_ref[...].astype(jnp.bfloat16)
    for j in range(n_tiles):
        @pl.when(i == 0)
        def _(j=j):
            if j + 1 < n_tiles:
                _start(j + 1)
            _wait_and_cast(j)

        sl = pl.ds(j * PFT, PFT)
        h = jnp.dot(xb, w1bf[:, sl], preferred_element_type=jnp.float32)
        h = jnp.maximum(h + b1_ref[:, sl], 0.0)
        hb[:, sl] = h.astype(jnp.bfloat16)
    out = jnp.dot(hb[...], w2bf[...], preferred_element_type=jnp.float32)
    out_ref[...] = out + b2_ref[...]


@functools.partial(jax.jit, static_argnames=())
def kernel(x, W1, b1, W2, b2):
    B, S, H = x.shape
    PF = W1.shape[1]
    M = B * S
    x2 = x.reshape(M, H)
    b1r = b1.reshape(1, PF)
    b2r = b2.reshape(1, H)

    out = pl.pallas_call(
        _ffn_kernel,
        grid=(M // BM,),
        in_specs=[
            pl.BlockSpec((BM, H), lambda i: (i, 0)),
            pl.BlockSpec(memory_space=pl.ANY),
            pl.BlockSpec((1, PF), lambda i: (0, 0)),
            pl.BlockSpec(memory_space=pl.ANY),
            pl.BlockSpec((1, H), lambda i: (0, 0)),
        ],
        out_specs=pl.BlockSpec((BM, H), lambda i: (i, 0)),
        out_shape=jax.ShapeDtypeStruct((M, H), jnp.float32),
        scratch_shapes=[
            pltpu.VMEM((H, PF), jnp.bfloat16),
            pltpu.VMEM((PF, H), jnp.bfloat16),
            pltpu.VMEM((BM, PF), jnp.bfloat16),
            pltpu.VMEM((2, H, PFT), jnp.float32),
            pltpu.VMEM((2, PFT, H), jnp.float32),
            pltpu.SemaphoreType.DMA((2,)),
            pltpu.SemaphoreType.DMA((2,)),
        ],
        compiler_params=pltpu.CompilerParams(
            dimension_semantics=("parallel",),
            vmem_limit_bytes=63 * 1024 * 1024,
        ),
    )(x2, W1, b1r, W2, b2r)
    return out.reshape(B, S, H)
